# baseline (speedup 1.0000x reference)
import functools

import jax
import jax.numpy as jnp
from jax import lax
from jax.experimental import pallas as pl
from jax.experimental.pallas import tpu as pltpu
from jax.experimental.pallas import tpu_sc as plsc

_NUM_WORKERS = 32
_CHUNK = 128


def kernel(point_id, cffs_scaled, cff_scales):
    B = point_id.shape[0]
    D = cffs_scaled.shape[1]
    L = 16
    b_per_w = B // _NUM_WORKERS
    n_chunks = b_per_w // _CHUNK
    elems_w = b_per_w * D
    rows_per_vec = L // D

    idx2d = point_id.astype(jnp.int32).reshape(B // _CHUNK, _CHUNK)
    scales16 = jnp.tile(cff_scales, rows_per_vec)

    mesh = plsc.VectorSubcoreMesh(core_axis_name="c", subcore_axis_name="s")

    @functools.partial(
        pl.kernel,
        mesh=mesh,
        compiler_params=pltpu.CompilerParams(
            use_tc_tiling_on_sc=False, needs_layout_passes=False
        ),
        out_type=jax.ShapeDtypeStruct((B * D,), jnp.float32),
        scratch_types=[
            pltpu.VMEM((n_chunks, _CHUNK), jnp.int32),
            pltpu.VMEM((b_per_w, D), jnp.float32),
            pltpu.VMEM((elems_w,), jnp.float32),
            pltpu.VMEM((L,), jnp.float32),
            pltpu.SemaphoreType.DMA,
        ],
    )
    def k(idx_hbm, table_hbm, scales_hbm, out_hbm, idx_v, rows_v, out_v,
          sc_v, sem):
        wid = lax.axis_index("s") * 2 + lax.axis_index("c")
        pltpu.sync_copy(idx_hbm.at[pl.ds(wid * n_chunks, n_chunks)], idx_v)
        pltpu.sync_copy(scales_hbm, sc_v)
        gathers = []
        for c in range(n_chunks):
            gathers.append(
                pltpu.async_copy(
                    table_hbm.at[idx_v.at[c]],
                    rows_v.at[pl.ds(c * _CHUNK, _CHUNK)],
                    sem,
                )
            )
        for g in gathers:
            g.wait()

        s = sc_v[...]
        iota = lax.iota(jnp.int32, L)
        col = lax.bitwise_and(iota, D - 1)
        row_off = lax.shift_right_logical(iota, 3)

        def body(i, _):
            r = row_off + i * rows_per_vec
            vals = plsc.load_gather(rows_v, [r, col])
            out_v[pl.ds(i * L, L)] = vals * s
            return 0

        lax.fori_loop(0, elems_w // L, body, 0)
        pltpu.sync_copy(out_v, out_hbm.at[pl.ds(wid * elems_w, elems_w)])

    out = k(idx2d, cffs_scaled, scales16)
    return out.reshape(B, D)


# speedup vs baseline: 8.5212x; 8.5212x over previous
import functools

import jax
import jax.numpy as jnp
from jax import lax
from jax.experimental import pallas as pl
from jax.experimental.pallas import tpu as pltpu
from jax.experimental.pallas import tpu_sc as plsc

_NUM_WORKERS = 32
_TW = 128
_CH = 32


def kernel(point_id, cffs_scaled, cff_scales):
    B = point_id.shape[0]
    D = cffs_scaled.shape[1]
    L = 16
    b_per_w = B // _NUM_WORKERS

    table_t = cffs_scaled.T
    idx2d = point_id.astype(jnp.int32).reshape(_NUM_WORKERS, b_per_w)

    mesh = plsc.VectorSubcoreMesh(core_axis_name="c", subcore_axis_name="s")

    @functools.partial(
        pl.kernel,
        mesh=mesh,
        compiler_params=pltpu.CompilerParams(needs_layout_passes=False),
        out_type=jax.ShapeDtypeStruct((D, B), jnp.float32),
        scratch_types=[
            pltpu.VMEM((1, b_per_w), jnp.int32),
            pltpu.VMEM((_CH * D, _TW), jnp.float32),
            pltpu.VMEM((D, b_per_w), jnp.float32),
            pltpu.VMEM((L,), jnp.float32),
            pltpu.SemaphoreType.DMA,
        ],
    )
    def k(idx_hbm, table_hbm, scales_hbm, out_hbm, idx_vm, blk_v, f_v, sc_v,
          sem):
        wid = lax.axis_index("s") * 2 + lax.axis_index("c")
        pltpu.sync_copy(idx_hbm.at[pl.ds(wid, 1)], idx_vm)
        pltpu.sync_copy(scales_hbm, sc_v.at[pl.ds(0, D)])

        s = sc_v[...]
        iota = lax.iota(jnp.int32, L)

        def chunk(g, _):
            base = g * _CH
            copies = []
            for v16 in range(_CH // L):
                vec = idx_vm[0, pl.ds(base + v16 * L, L)]
                gbase = lax.shift_left(
                    lax.shift_right_logical(vec, 7), 7
                )
                for j in range(L):
                    off = pl.multiple_of(gbase[j], _TW)
                    copies.append(
                        pltpu.async_copy(
                            table_hbm.at[:, pl.ds(off, _TW)],
                            blk_v.at[pl.ds((v16 * L + j) * D, D), :],
                            sem,
                        )
                    )
            for cp in copies:
                cp.wait()
            for v16 in range(_CH // L):
                idx16 = idx_vm[0, pl.ds(base + v16 * L, L)]
                lanes = lax.bitwise_and(idx16, _TW - 1)
                rows0 = (v16 * L + iota) * D
                for c in range(D):
                    vals = plsc.load_gather(blk_v, [rows0 + c, lanes])
                    f_v[c, pl.ds(base + v16 * L, L)] = (
                        vals * lax.broadcast_in_dim(s[c], (L,), ())
                    )
            return 0

        lax.fori_loop(0, b_per_w // _CH, chunk, 0)
        pltpu.sync_copy(f_v, out_hbm.at[:, pl.ds(wid * b_per_w, b_per_w)])

    out = k(idx2d, table_t, cff_scales)
    return out.T


# speedup vs baseline: 9.0881x; 1.0665x over previous
import functools

import jax
import jax.numpy as jnp
from jax import lax
from jax.experimental import pallas as pl
from jax.experimental.pallas import tpu as pltpu
from jax.experimental.pallas import tpu_sc as plsc

_NUM_WORKERS = 32
_TW = 128
_CH = 32


def kernel(point_id, cffs_scaled, cff_scales):
    B = point_id.shape[0]
    D = cffs_scaled.shape[1]
    L = 16
    b_per_w = B // _NUM_WORKERS

    table_t = cffs_scaled.T
    idx2d = point_id.astype(jnp.int32).reshape(_NUM_WORKERS, b_per_w)

    mesh = plsc.VectorSubcoreMesh(core_axis_name="c", subcore_axis_name="s")

    @functools.partial(
        pl.kernel,
        mesh=mesh,
        compiler_params=pltpu.CompilerParams(needs_layout_passes=False),
        out_type=jax.ShapeDtypeStruct((D, B), jnp.float32),
        scratch_types=[
            pltpu.VMEM((1, b_per_w), jnp.int32),
            pltpu.VMEM((_CH * D, _TW), jnp.float32),
            pltpu.VMEM((_CH * D, _TW), jnp.float32),
            pltpu.VMEM((D, b_per_w), jnp.float32),
            pltpu.VMEM((L,), jnp.float32),
            pltpu.SemaphoreType.DMA,
            pltpu.SemaphoreType.DMA,
        ],
    )
    def k(idx_hbm, table_hbm, scales_hbm, out_hbm, idx_vm, blk_a, blk_b,
          f_v, sc_v, sem_a, sem_b):
        wid = lax.axis_index("s") * 2 + lax.axis_index("c")
        pltpu.sync_copy(idx_hbm.at[pl.ds(wid, 1)], idx_vm)
        pltpu.sync_copy(scales_hbm, sc_v.at[pl.ds(0, D)])

        s = sc_v[...]
        iota = lax.iota(jnp.int32, L)
        n_chunks = b_per_w // _CH

        def fire(g, blk, sem):
            base = g * _CH
            copies = []
            for v16 in range(_CH // L):
                vec = idx_vm[0, pl.ds(base + v16 * L, L)]
                gbase = lax.shift_left(
                    lax.shift_right_logical(vec, 7), 7
                )
                for j in range(L):
                    off = pl.multiple_of(gbase[j], _TW)
                    copies.append(
                        pltpu.async_copy(
                            table_hbm.at[:, pl.ds(off, _TW)],
                            blk.at[pl.ds((v16 * L + j) * D, D), :],
                            sem,
                        )
                    )
            return copies

        def extract(g, blk):
            base = g * _CH
            for v16 in range(_CH // L):
                idx16 = idx_vm[0, pl.ds(base + v16 * L, L)]
                lanes = lax.bitwise_and(idx16, _TW - 1)
                rows0 = (v16 * L + iota) * D
                for c in range(D):
                    vals = plsc.load_gather(blk, [rows0 + c, lanes])
                    f_v[c, pl.ds(base + v16 * L, L)] = (
                        vals * lax.broadcast_in_dim(s[c], (L,), ())
                    )

        def pair(h, _):
            ga = h * 2
            gb = h * 2 + 1
            ca = fire(ga, blk_a, sem_a)
            cb = fire(gb, blk_b, sem_b)
            for cp in ca:
                cp.wait()
            extract(ga, blk_a)
            for cp in cb:
                cp.wait()
            extract(gb, blk_b)
            return 0

        lax.fori_loop(0, n_chunks // 2, pair, 0)
        pltpu.sync_copy(f_v, out_hbm.at[:, pl.ds(wid * b_per_w, b_per_w)])

    out = k(idx2d, table_t, cff_scales)
    return out.T
